# baseline (speedup 1.0000x reference)
import jax
import jax.numpy as jnp
from jax import lax
from jax.experimental import pallas as pl
from jax.experimental.pallas import tpu as pltpu
from jax.experimental.pallas import tpu_sc as plsc

B = 4096
T = 20
V = 1000
H = 1024

NC = 2
NS = 16
NW = NC * NS
SENT_W = B // NW
VC = 1000
HZ = 64


def _sc_body(x_hbm, z_h_hbm, sent_hbm, hidden_hbm,
             tok_v, sent_v, zbuf, hsem):
    cid = lax.axis_index("c")
    sid = lax.axis_index("s")
    wid = sid * NC + cid
    s0 = wid * SENT_W
    lane = lax.iota(jnp.int32, 16)

    pltpu.sync_copy(z_h_hbm, zbuf)

    for t in range(T):
        for k in range(SENT_W // HZ):
            pltpu.async_copy(zbuf, hidden_hbm.at[t, pl.ds(s0 + k * HZ, HZ)],
                             hsem)

    pltpu.sync_copy(x_hbm.at[:, pl.ds(s0, SENT_W)], tok_v)

    for j in range(SENT_W // 16):
        srow = j * 16 + lane

        def tstep(t, seen, srow=srow):
            tcol = jnp.full((16,), 0, jnp.int32) + t
            toki = plsc.load_gather(tok_v, [tcol, srow]).astype(jnp.int32)
            seen = seen | (toki == 0).astype(jnp.int32)
            out = jnp.where(seen == 1, 0, toki)
            plsc.store_scatter(sent_v, [tcol, srow], out)
            return seen

        lax.fori_loop(0, T, tstep, jnp.zeros((16,), jnp.int32))
    pltpu.sync_copy(sent_v, sent_hbm.at[:, pl.ds(s0, SENT_W)])

    for t in range(T):
        for k in range(SENT_W // HZ):
            pltpu.make_async_copy(zbuf,
                                  hidden_hbm.at[t, pl.ds(s0 + k * HZ, HZ)],
                                  hsem).wait()


def _tc_onehot_body(tok_ref, o_ref):
    t = pl.program_id(0)
    vc = pl.program_id(1)
    tok = tok_ref[t, :].astype(jnp.int32)
    vio = lax.broadcasted_iota(jnp.int32, (VC, B), 0) + vc * VC
    o_ref[0] = jnp.where(vio == tok[None, :], 1.0, 0.0)


@jax.jit
def _run(x_tb, z_h):
    mesh = plsc.VectorSubcoreMesh(core_axis_name="c", subcore_axis_name="s")
    sent_tb, hidden_tbh = pl.kernel(
        _sc_body,
        out_type=[
            jax.ShapeDtypeStruct((T, B), jnp.int32),
            jax.ShapeDtypeStruct((T, B, H), jnp.float32),
        ],
        mesh=mesh,
        scratch_types=[
            pltpu.VMEM((T, SENT_W), jnp.float32),
            pltpu.VMEM((T, SENT_W), jnp.int32),
            pltpu.VMEM((HZ, H), jnp.float32),
            pltpu.SemaphoreType.DMA,
        ],
        compiler_params=pltpu.CompilerParams(
            needs_layout_passes=False, use_tc_tiling_on_sc=True),
        name="eos_hidden_sc",
    )(x_tb, z_h)

    logits_tvb = pl.pallas_call(
        _tc_onehot_body,
        out_shape=jax.ShapeDtypeStruct((T, V, B), jnp.float32),
        grid=(T, V // VC),
        in_specs=[pl.BlockSpec((T, B), lambda t, v: (0, 0))],
        out_specs=pl.BlockSpec((1, VC, B), lambda t, v: (t, v, 0)),
        name="onehot_tc",
    )(x_tb)
    return sent_tb, logits_tvb, hidden_tbh


def kernel(x):
    x_tb = jnp.transpose(x, (1, 0))
    z_h = jnp.zeros((HZ, H), jnp.float32)
    sent_tb, logits_tvb, hidden_tbh = _run(x_tb, z_h)
    return (jnp.transpose(sent_tb, (1, 0)),
            jnp.transpose(logits_tvb, (2, 0, 1)),
            jnp.transpose(hidden_tbh, (1, 0, 2)))


# speedup vs baseline: 1.0253x; 1.0253x over previous
import jax
import jax.numpy as jnp
from jax import lax
from jax.experimental import pallas as pl
from jax.experimental.pallas import tpu as pltpu
from jax.experimental.pallas import tpu_sc as plsc

B = 4096
T = 20
V = 1000
H = 1024

NC = 2
NS = 16
NW = NC * NS
SENT_W = B // NW
VC = 200
HZ = 64


def _sc_body(x_hbm, z_h_hbm, sent_hbm, hidden_hbm,
             tok_v, sent_v, zbuf, hsem):
    cid = lax.axis_index("c")
    sid = lax.axis_index("s")
    wid = sid * NC + cid
    s0 = wid * SENT_W
    lane = lax.iota(jnp.int32, 16)

    pltpu.sync_copy(z_h_hbm, zbuf)

    for t in range(T):
        for k in range(SENT_W // HZ):
            pltpu.async_copy(zbuf, hidden_hbm.at[t, pl.ds(s0 + k * HZ, HZ)],
                             hsem)

    pltpu.sync_copy(x_hbm.at[:, pl.ds(s0, SENT_W)], tok_v)

    for j in range(SENT_W // 16):
        srow = j * 16 + lane

        def tstep(t, seen, srow=srow):
            tcol = jnp.full((16,), 0, jnp.int32) + t
            toki = plsc.load_gather(tok_v, [tcol, srow]).astype(jnp.int32)
            seen = seen | (toki == 0).astype(jnp.int32)
            out = jnp.where(seen == 1, 0, toki)
            plsc.store_scatter(sent_v, [tcol, srow], out)
            return seen

        lax.fori_loop(0, T, tstep, jnp.zeros((16,), jnp.int32))
    pltpu.sync_copy(sent_v, sent_hbm.at[:, pl.ds(s0, SENT_W)])

    for t in range(T):
        for k in range(SENT_W // HZ):
            pltpu.make_async_copy(zbuf,
                                  hidden_hbm.at[t, pl.ds(s0 + k * HZ, HZ)],
                                  hsem).wait()


def _tc_onehot_body(tok_ref, o_ref):
    t = pl.program_id(0)
    vc = pl.program_id(1)
    tok = tok_ref[t, :].astype(jnp.int32)
    vio = lax.broadcasted_iota(jnp.int32, (VC, B), 0) + vc * VC
    o_ref[0] = jnp.where(vio == tok[None, :], 1.0, 0.0)


@jax.jit
def _run(x_tb, z_h):
    mesh = plsc.VectorSubcoreMesh(core_axis_name="c", subcore_axis_name="s")
    sent_tb, hidden_tbh = pl.kernel(
        _sc_body,
        out_type=[
            jax.ShapeDtypeStruct((T, B), jnp.int32),
            jax.ShapeDtypeStruct((T, B, H), jnp.float32),
        ],
        mesh=mesh,
        scratch_types=[
            pltpu.VMEM((T, SENT_W), jnp.float32),
            pltpu.VMEM((T, SENT_W), jnp.int32),
            pltpu.VMEM((HZ, H), jnp.float32),
            pltpu.SemaphoreType.DMA,
        ],
        compiler_params=pltpu.CompilerParams(
            needs_layout_passes=False, use_tc_tiling_on_sc=True),
        name="eos_hidden_sc",
    )(x_tb, z_h)

    logits_tvb = pl.pallas_call(
        _tc_onehot_body,
        out_shape=jax.ShapeDtypeStruct((T, V, B), jnp.float32),
        grid=(T, V // VC),
        in_specs=[pl.BlockSpec((T, B), lambda t, v: (0, 0))],
        out_specs=pl.BlockSpec((1, VC, B), lambda t, v: (t, v, 0)),
        name="onehot_tc",
    )(x_tb)
    return sent_tb, logits_tvb, hidden_tbh


def kernel(x):
    x_tb = jnp.transpose(x, (1, 0))
    z_h = jnp.zeros((HZ, H), jnp.float32)
    sent_tb, logits_tvb, hidden_tbh = _run(x_tb, z_h)
    return (jnp.transpose(sent_tb, (1, 0)),
            jnp.transpose(logits_tvb, (2, 0, 1)),
            jnp.transpose(hidden_tbh, (1, 0, 2)))


# speedup vs baseline: 1.0301x; 1.0047x over previous
import jax
import jax.numpy as jnp
from jax import lax
from jax.experimental import pallas as pl
from jax.experimental.pallas import tpu as pltpu
from jax.experimental.pallas import tpu_sc as plsc

B = 4096
T = 20
V = 1000
H = 1024

NC = 2
NS = 16
NW = NC * NS
SENT_W = B // NW
VC = 200
HZ = 64


def _sc_body(x_hbm, z_h_hbm, sent_hbm, hidden_hbm,
             tok_v, sent_v, zbuf, hsem):
    cid = lax.axis_index("c")
    sid = lax.axis_index("s")
    wid = sid * NC + cid
    s0 = wid * SENT_W
    lane = lax.iota(jnp.int32, 16)

    pltpu.sync_copy(x_hbm.at[:, pl.ds(s0, SENT_W)], tok_v)

    pltpu.sync_copy(z_h_hbm, zbuf)

    for t in range(T):
        for k in range(SENT_W // HZ):
            pltpu.async_copy(zbuf, hidden_hbm.at[t, pl.ds(s0 + k * HZ, HZ)],
                             hsem)

    for j in range(SENT_W // 16):
        srow = j * 16 + lane

        def tstep(t, seen, srow=srow):
            tcol = jnp.full((16,), 0, jnp.int32) + t
            toki = plsc.load_gather(tok_v, [tcol, srow]).astype(jnp.int32)
            seen = seen | (toki == 0).astype(jnp.int32)
            out = jnp.where(seen == 1, 0, toki)
            plsc.store_scatter(sent_v, [tcol, srow], out)
            return seen

        lax.fori_loop(0, T, tstep, jnp.zeros((16,), jnp.int32))
    pltpu.sync_copy(sent_v, sent_hbm.at[:, pl.ds(s0, SENT_W)])

    for t in range(T):
        for k in range(SENT_W // HZ):
            pltpu.make_async_copy(zbuf,
                                  hidden_hbm.at[t, pl.ds(s0 + k * HZ, HZ)],
                                  hsem).wait()


def _tc_onehot_body(tok_ref, o_ref):
    t = pl.program_id(0)
    vc = pl.program_id(1)
    tok = tok_ref[t, :].astype(jnp.int32)
    vio = lax.broadcasted_iota(jnp.int32, (VC, B), 0) + vc * VC
    o_ref[0] = jnp.where(vio == tok[None, :], 1.0, 0.0)


@jax.jit
def _run(x_tb, z_h):
    mesh = plsc.VectorSubcoreMesh(core_axis_name="c", subcore_axis_name="s")
    sent_tb, hidden_tbh = pl.kernel(
        _sc_body,
        out_type=[
            jax.ShapeDtypeStruct((T, B), jnp.int32),
            jax.ShapeDtypeStruct((T, B, H), jnp.float32),
        ],
        mesh=mesh,
        scratch_types=[
            pltpu.VMEM((T, SENT_W), jnp.float32),
            pltpu.VMEM((T, SENT_W), jnp.int32),
            pltpu.VMEM((HZ, H), jnp.float32),
            pltpu.SemaphoreType.DMA,
        ],
        compiler_params=pltpu.CompilerParams(
            needs_layout_passes=False, use_tc_tiling_on_sc=True),
        name="eos_hidden_sc",
    )(x_tb, z_h)

    logits_tvb = pl.pallas_call(
        _tc_onehot_body,
        out_shape=jax.ShapeDtypeStruct((T, V, B), jnp.float32),
        grid=(T, V // VC),
        in_specs=[pl.BlockSpec((T, B), lambda t, v: (0, 0))],
        out_specs=pl.BlockSpec((1, VC, B), lambda t, v: (t, v, 0)),
        name="onehot_tc",
    )(x_tb)
    return sent_tb, logits_tvb, hidden_tbh


def kernel(x):
    x_tb = jnp.transpose(x, (1, 0))
    z_h = jnp.zeros((HZ, H), jnp.float32)
    sent_tb, logits_tvb, hidden_tbh = _run(x_tb, z_h)
    return (jnp.transpose(sent_tb, (1, 0)),
            jnp.transpose(logits_tvb, (2, 0, 1)),
            jnp.transpose(hidden_tbh, (1, 0, 2)))
